# baseline (speedup 1.0000x reference)
import functools

import jax
import jax.numpy as jnp
from jax import lax
from jax.experimental import pallas as pl
from jax.experimental.pallas import tpu as pltpu
from jax.experimental.pallas import tpu_sc as plsc

NUM_EMB = 1_000_000
DIM = 64
B_TOTAL = 16384 * 20
LANE = 128
N_WORKERS = 32
ROWS_TOTAL = B_TOTAL // LANE
ROWS_PER_W = ROWS_TOTAL // N_WORKERS
KCH = 4
N_CHUNKS = ROWS_PER_W // KCH

CW = 128
NCH = NUM_EMB // CW
CH_PER_W = NCH // N_WORKERS
NCH_REM = NCH - CH_PER_W * N_WORKERS
N_PAIR = NUM_EMB // 2


def _sc_transpose(at, tail_pairs):
    mesh = plsc.VectorSubcoreMesh(core_axis_name="c", subcore_axis_name="s")

    @functools.partial(
        pl.kernel,
        out_type=jax.ShapeDtypeStruct((N_PAIR, 128), jnp.float32),
        mesh=mesh,
        scratch_types=[
            pltpu.VMEM((DIM, CW), jnp.float32),
            pltpu.VMEM((DIM, CW), jnp.float32),
            pltpu.VMEM((CW // 2, 128), jnp.float32),
            pltpu.VMEM((CW // 2, 128), jnp.float32),
            pltpu.SemaphoreType.DMA,
            pltpu.SemaphoreType.DMA,
            pltpu.SemaphoreType.DMA,
            pltpu.SemaphoreType.DMA,
        ],
        compiler_params=pltpu.CompilerParams(
            use_tc_tiling_on_sc=True, needs_layout_passes=False
        ),
    )
    def k(at_hbm, tail_hbm, lin_hbm, in0, in1, ob0, ob1, si0, si1, so0, so1):
        wid = lax.axis_index("s") * 2 + lax.axis_index("c")
        ch0 = wid * CH_PER_W
        ins = (in0, in1)
        obs = (ob0, ob1)
        sis = (si0, si1)
        sos = (so0, so1)
        iota = lax.iota(jnp.int32, 16)

        def fire_in(g, b):
            c0 = pl.multiple_of((ch0 + g) * CW, CW)
            pltpu.async_copy(at_hbm.at[:, pl.ds(c0, CW)], ins[b], sis[b])

        def wait_in(g, b):
            c0 = pl.multiple_of((ch0 + g) * CW, CW)
            pltpu.make_async_copy(
                at_hbm.at[:, pl.ds(c0, CW)], ins[b], sis[b]
            ).wait()

        row_vecs = [
            lax.shift_right_logical(iota + cb, jnp.full((16,), 1, jnp.int32))
            for cb in range(0, CW, 16)
        ]
        col_vecs = [
            lax.bitwise_and(iota + cb, jnp.full((16,), 1, jnp.int32)) * 64
            for cb in range(0, CW, 16)
        ]

        def transpose(b):
            @plsc.parallel_loop(0, DIM, step=1, unroll=16)
            def _(d):
                for cbi in range(CW // 16):
                    v = ins[b][d, pl.ds(cbi * 16, 16)]
                    plsc.store_scatter(
                        obs[b], [row_vecs[cbi], col_vecs[cbi] + d], v
                    )

        def fire_out(g, b):
            r0 = pl.multiple_of((ch0 + g) * (CW // 2), CW // 2)
            pltpu.async_copy(obs[b], lin_hbm.at[pl.ds(r0, CW // 2)], sos[b])

        def wait_out(g, b):
            r0 = pl.multiple_of((ch0 + g) * (CW // 2), CW // 2)
            pltpu.make_async_copy(
                obs[b], lin_hbm.at[pl.ds(r0, CW // 2)], sos[b]
            ).wait()

        fire_in(0, 0)

        def outer(gg, carry):
            for b in range(2):
                g = gg * 2 + b

                @pl.when(g + 1 < CH_PER_W)
                def _():
                    fire_in(g + 1, 1 - b)

                wait_in(g, b)

                @pl.when(g >= 2)
                def _():
                    wait_out(g - 2, b)

                transpose(b)
                fire_out(g, b)
            return carry

        lax.fori_loop(0, CH_PER_W // 2, outer, 0)
        wait_out(CH_PER_W - 2, 0)
        wait_out(CH_PER_W - 1, 1)

        @pl.when(wid < NCH_REM)
        def _():
            c0 = pl.multiple_of((CH_PER_W * N_WORKERS + wid) * CW, CW)
            r0 = pl.multiple_of(
                (CH_PER_W * N_WORKERS + wid) * (CW // 2), CW // 2
            )
            pltpu.sync_copy(at_hbm.at[:, pl.ds(c0, CW)], in0)
            transpose(0)
            pltpu.sync_copy(ob0, lin_hbm.at[pl.ds(r0, CW // 2)])

        @pl.when(wid == NCH_REM)
        def _():
            pltpu.sync_copy(tail_hbm, ob1.at[pl.ds(0, 32)])
            pltpu.sync_copy(
                ob1.at[pl.ds(0, 32)], lin_hbm.at[pl.ds(N_PAIR - 32, 32)]
            )

    return k(at, tail_pairs)


def _sc_gather(x2, lin):
    mesh = plsc.VectorSubcoreMesh(core_axis_name="c", subcore_axis_name="s")

    @functools.partial(
        pl.kernel,
        out_type=jax.ShapeDtypeStruct((ROWS_TOTAL, LANE, DIM), jnp.float32),
        mesh=mesh,
        scratch_types=[
            pltpu.VMEM((KCH, LANE), jnp.int32),
            pltpu.VMEM((KCH, LANE), jnp.int32),
            pltpu.VMEM((KCH, LANE, DIM), jnp.float32),
            pltpu.VMEM((KCH, LANE, DIM), jnp.float32),
            pltpu.SemaphoreType.DMA,
            pltpu.SemaphoreType.DMA,
        ],
        compiler_params=pltpu.CompilerParams(
            use_tc_tiling_on_sc=False, needs_layout_passes=False
        ),
    )
    def k(x_hbm, tab_hbm, out_hbm, idx0, idx1, rows0, rows1, sem0, sem1):
        wid = lax.axis_index("s") * 2 + lax.axis_index("c")
        row0 = wid * ROWS_PER_W
        tab = tab_hbm
        idx_b = (idx0, idx1)
        rows_b = (rows0, rows1)
        sem_b = (sem0, sem1)

        def fire(g, b):
            r = row0 + g * KCH
            pltpu.sync_copy(x_hbm.at[pl.ds(r, KCH)], idx_b[b])
            for j in range(KCH):
                pltpu.async_copy(
                    tab.at[idx_b[b].at[j]], rows_b[b].at[j], sem_b[b]
                )

        def drain_store(g, b):
            for j in range(KCH):
                pltpu.make_async_copy(
                    tab.at[idx_b[b].at[j]], rows_b[b].at[j], sem_b[b]
                ).wait()
            pltpu.sync_copy(rows_b[b], out_hbm.at[pl.ds(row0 + g * KCH, KCH)])

        fire(0, 0)

        def outer(gg, carry):
            for b in range(2):
                g = gg * 2 + b

                @pl.when(g + 1 < N_CHUNKS)
                def _():
                    fire(g + 1, 1 - b)

                drain_store(g, b)
            return carry

        lax.fori_loop(0, N_CHUNKS // 2, outer, 0)

    return k(x2, lin)


def kernel(x, A):
    x2 = x.T.reshape(ROWS_TOTAL, LANE).astype(jnp.int32)
    tail_pairs = A[NCH * CW:, :].reshape(32, 128)
    lin = _sc_transpose(A.T, tail_pairs).reshape(NUM_EMB, DIM)
    out = _sc_gather(x2, lin)
    return out.reshape(20, 16384, DIM).transpose(1, 0, 2)


# speedup vs baseline: 1.3088x; 1.3088x over previous
import functools

import jax
import jax.numpy as jnp
from jax import lax
from jax.experimental import pallas as pl
from jax.experimental.pallas import tpu as pltpu
from jax.experimental.pallas import tpu_sc as plsc

NUM_EMB = 1_000_000
DIM = 64
B_TOTAL = 16384 * 20
LANE = 128
N_WORKERS = 32
ROWS_TOTAL = B_TOTAL // LANE
ROWS_PER_W = ROWS_TOTAL // N_WORKERS
KCH = 4
N_CHUNKS = ROWS_PER_W // KCH


def _sc_gather(x2, table):
    mesh = plsc.VectorSubcoreMesh(core_axis_name="c", subcore_axis_name="s")

    @functools.partial(
        pl.kernel,
        out_type=jax.ShapeDtypeStruct((ROWS_TOTAL, LANE, DIM), jnp.float32),
        mesh=mesh,
        scratch_types=[
            pltpu.VMEM((KCH, LANE), jnp.int32),
            pltpu.VMEM((KCH, LANE), jnp.int32),
            pltpu.VMEM((KCH, LANE, DIM), jnp.float32),
            pltpu.VMEM((KCH, LANE, DIM), jnp.float32),
            pltpu.SemaphoreType.DMA,
            pltpu.SemaphoreType.DMA,
        ],
        compiler_params=pltpu.CompilerParams(use_tc_tiling_on_sc=False),
    )
    def k(x_hbm, tab_hbm, out_hbm, idx0, idx1, rows0, rows1, sem0, sem1):
        wid = lax.axis_index("s") * 2 + lax.axis_index("c")
        row0 = wid * ROWS_PER_W
        idx_b = (idx0, idx1)
        rows_b = (rows0, rows1)
        sem_b = (sem0, sem1)

        def fire(g, b):
            r = row0 + g * KCH
            pltpu.sync_copy(x_hbm.at[pl.ds(r, KCH)], idx_b[b])
            for j in range(KCH):
                pltpu.async_copy(
                    tab_hbm.at[idx_b[b].at[j]], rows_b[b].at[j], sem_b[b]
                )

        def drain_store(g, b):
            for j in range(KCH):
                pltpu.make_async_copy(
                    tab_hbm.at[idx_b[b].at[j]], rows_b[b].at[j], sem_b[b]
                ).wait()
            pltpu.sync_copy(rows_b[b], out_hbm.at[pl.ds(row0 + g * KCH, KCH)])

        fire(0, 0)

        def outer(gg, carry):
            for b in range(2):
                g = gg * 2 + b

                @pl.when(g + 1 < N_CHUNKS)
                def _():
                    fire(g + 1, 1 - b)

                drain_store(g, b)
            return carry

        lax.fori_loop(0, N_CHUNKS // 2, outer, 0)

    return k(x2, table)


def kernel(x, A):
    x2 = x.T.reshape(ROWS_TOTAL, LANE).astype(jnp.int32)
    out = _sc_gather(x2, A)
    return out.reshape(20, 16384, DIM).transpose(1, 0, 2)
